# Initial kernel scaffold; baseline (speedup 1.0000x reference)
import jax
import jax.numpy as jnp
from jax.experimental import pallas as pl


def kernel(input_question, input_answer, embeddings):
    raise NotImplementedError("write your pallas kernel here")


# speedup vs baseline: 2.2624x; 2.2624x over previous
import functools

import jax
import jax.numpy as jnp
from jax import lax
from jax.experimental import pallas as pl
from jax.experimental.pallas import tpu as pltpu
from jax.experimental.pallas import tpu_sc as plsc

D = 128
B = 4096
QL = 20
AL = 50
NQ = B * QL
NA = B * AL
NW = 32
CH = 128
QCH = NQ // (NW * CH)
ACH = NA // (NW * CH)


def _sc_gather(idx_q, idx_a, table):
    mesh = plsc.VectorSubcoreMesh(core_axis_name="c", subcore_axis_name="s")

    @functools.partial(
        pl.kernel,
        mesh=mesh,
        out_type=[
            jax.ShapeDtypeStruct((NQ, D), jnp.float32),
            jax.ShapeDtypeStruct((NA, D), jnp.float32),
        ],
        scratch_types=[
            pltpu.VMEM((CH,), jnp.int32),
            pltpu.VMEM((CH, D), jnp.float32),
            pltpu.SemaphoreType.DMA,
        ],
    )
    def k(idx_q_hbm, idx_a_hbm, table_hbm, out_q_hbm, out_a_hbm,
          idx_v, rows_v, sem):
        wid = lax.axis_index("s") * 2 + lax.axis_index("c")

        def run(idx_hbm, out_hbm, nchunks):
            base = wid * nchunks * CH

            def body(i, carry):
                off = base + i * CH
                pltpu.sync_copy(idx_hbm.at[pl.ds(off, CH)], idx_v)
                pltpu.async_copy(table_hbm.at[idx_v], rows_v, sem).wait()
                pltpu.sync_copy(rows_v, out_hbm.at[pl.ds(off, CH)])
                return carry

            lax.fori_loop(0, nchunks, body, 0)

        run(idx_q_hbm, out_q_hbm, QCH)
        run(idx_a_hbm, out_a_hbm, ACH)

    return k(idx_q, idx_a, table)


def _masks(iq, ia):
    def body(q_ref, a_ref, mq_ref, ma_ref):
        for ref, out in ((q_ref, mq_ref), (a_ref, ma_ref)):
            x = ref[...].astype(jnp.float32)
            m = jnp.max(x, axis=1, keepdims=True)
            out[...] = jnp.ceil(x / m)

    nb = 8
    bb = B // nb
    return pl.pallas_call(
        body,
        grid=(nb,),
        in_specs=[
            pl.BlockSpec((bb, QL), lambda i: (i, 0)),
            pl.BlockSpec((bb, AL), lambda i: (i, 0)),
        ],
        out_specs=[
            pl.BlockSpec((bb, QL), lambda i: (i, 0)),
            pl.BlockSpec((bb, AL), lambda i: (i, 0)),
        ],
        out_shape=[
            jax.ShapeDtypeStruct((B, QL), jnp.float32),
            jax.ShapeDtypeStruct((B, AL), jnp.float32),
        ],
    )(iq, ia)


def kernel(input_question, input_answer, embeddings):
    eq, ea = _sc_gather(
        input_question.reshape(-1), input_answer.reshape(-1), embeddings)
    mq, ma = _masks(input_question, input_answer)
    return eq.reshape(B, QL, D), ea.reshape(B, AL, D), mq, ma


# speedup vs baseline: 2.7357x; 1.2092x over previous
import functools

import jax
import jax.numpy as jnp
from jax import lax
from jax.experimental import pallas as pl
from jax.experimental.pallas import tpu as pltpu
from jax.experimental.pallas import tpu_sc as plsc

D = 128
B = 4096
QL = 20
AL = 50
NQ = B * QL
NA = B * AL
NW = 32
CH = 128
QCH = NQ // (NW * CH)
ACH = NA // (NW * CH)


NB = 5


def _sc_gather(idx_q, idx_a, table):
    mesh = plsc.VectorSubcoreMesh(core_axis_name="c", subcore_axis_name="s")

    @functools.partial(
        pl.kernel,
        mesh=mesh,
        out_type=[
            jax.ShapeDtypeStruct((NQ, D), jnp.float32),
            jax.ShapeDtypeStruct((NA, D), jnp.float32),
        ],
        scratch_types=(
            [pltpu.VMEM((QCH * CH,), jnp.int32),
             pltpu.VMEM((ACH * CH,), jnp.int32)]
            + [pltpu.VMEM((CH, D), jnp.float32) for _ in range(NB)]
            + [pltpu.SemaphoreType.DMA for _ in range(2 * NB)]
        ),
    )
    def k(idx_q_hbm, idx_a_hbm, table_hbm, out_q_hbm, out_a_hbm,
          idx_q_v, idx_a_v, *scratch):
        rows = scratch[:NB]
        gsem = scratch[NB:2 * NB]
        ssem = scratch[2 * NB:]
        wid = lax.axis_index("s") * 2 + lax.axis_index("c")

        pltpu.sync_copy(idx_q_hbm.at[pl.ds(wid * QCH * CH, QCH * CH)], idx_q_v)
        pltpu.sync_copy(idx_a_hbm.at[pl.ds(wid * ACH * CH, ACH * CH)], idx_a_v)

        def run(idx_v, out_hbm, nchunks, first):
            base = wid * nchunks * CH

            def body(t, carry):
                descs = []
                for b in range(NB):
                    c = t * NB + b
                    if not first:
                        pltpu.make_async_copy(
                            rows[b], out_hbm.at[pl.ds(0, CH)], ssem[b]).wait()
                    else:
                        @pl.when(t > 0)
                        def _():
                            pltpu.make_async_copy(
                                rows[b], out_hbm.at[pl.ds(0, CH)],
                                ssem[b]).wait()
                    descs.append(pltpu.async_copy(
                        table_hbm.at[idx_v.at[pl.ds(c * CH, CH)]],
                        rows[b], gsem[b]))
                for b in range(NB):
                    descs[b].wait()
                    pltpu.async_copy(
                        rows[b],
                        out_hbm.at[pl.ds(base + (t * NB + b) * CH, CH)],
                        ssem[b])
                return carry

            lax.fori_loop(0, nchunks // NB, body, 0, unroll=False)

        run(idx_q_v, out_q_hbm, QCH, first=True)
        run(idx_a_v, out_a_hbm, ACH, first=False)
        for b in range(NB):
            pltpu.make_async_copy(
                rows[b], out_a_hbm.at[pl.ds(0, CH)], ssem[b]).wait()

    return k(idx_q, idx_a, table)


def _masks(iq, ia):
    def body(q_ref, a_ref, mq_ref, ma_ref):
        for ref, out in ((q_ref, mq_ref), (a_ref, ma_ref)):
            x = ref[...].astype(jnp.float32)
            m = jnp.max(x, axis=1, keepdims=True)
            out[...] = jnp.ceil(x / m)

    nb = 8
    bb = B // nb
    return pl.pallas_call(
        body,
        grid=(nb,),
        in_specs=[
            pl.BlockSpec((bb, QL), lambda i: (i, 0)),
            pl.BlockSpec((bb, AL), lambda i: (i, 0)),
        ],
        out_specs=[
            pl.BlockSpec((bb, QL), lambda i: (i, 0)),
            pl.BlockSpec((bb, AL), lambda i: (i, 0)),
        ],
        out_shape=[
            jax.ShapeDtypeStruct((B, QL), jnp.float32),
            jax.ShapeDtypeStruct((B, AL), jnp.float32),
        ],
    )(iq, ia)


def kernel(input_question, input_answer, embeddings):
    eq, ea = _sc_gather(
        input_question.reshape(-1), input_answer.reshape(-1), embeddings)
    mq, ma = _masks(input_question, input_answer)
    return eq.reshape(B, QL, D), ea.reshape(B, AL, D), mq, ma
